# baseline (speedup 1.0000x reference)
import jax
import jax.numpy as jnp
from jax import lax
from jax.experimental import pallas as pl
from jax.experimental.pallas import tpu as pltpu
from jax.experimental.pallas import tpu_sc as plsc

EMBED = 64
B = 16384
NC, NS, L = 2, 16, 16
NW = NC * NS
ROWS = B // NW
CHUNK = 128
NCHUNK = ROWS // CHUNK


def _body(pidx_hbm, aidx_hbm, bidx_hbm, paths_hbm, ents_hbm,
          pos_out_hbm, neg_out_hbm,
          idxp_v, idxa_v, idxb_v, path_v, pos_v, neg_v,
          pos_s, neg_s, sem):
    wid = lax.axis_index("s") * NC + lax.axis_index("c")
    base = wid * ROWS

    pltpu.sync_copy(pidx_hbm.at[wid], idxp_v)
    pltpu.sync_copy(aidx_hbm.at[wid], idxa_v)
    pltpu.sync_copy(bidx_hbm.at[wid], idxb_v)

    copies = []
    for j in range(NCHUNK):
        dst = pl.ds(j * CHUNK, CHUNK)
        copies.append(pltpu.async_copy(paths_hbm.at[idxp_v.at[j]],
                                       path_v.at[dst], sem))
        copies.append(pltpu.async_copy(ents_hbm.at[idxa_v.at[j]],
                                       pos_v.at[dst], sem))
        copies.append(pltpu.async_copy(ents_hbm.at[idxb_v.at[j]],
                                       neg_v.at[dst], sem))
    for c in copies:
        c.wait()

    iota = lax.iota(jnp.int32, L)
    perms = [iota ^ s for s in (8, 4, 2, 1)]

    def fold(v):
        for p in perms:
            v = v + v.at[p].get(mode="promise_in_bounds")
        return v

    def group(g, carry):
        posvec = jnp.zeros((L,), jnp.float32)
        negvec = jnp.zeros((L,), jnp.float32)
        for r in range(L):
            i = g * L + r
            ap = jnp.zeros((L,), jnp.float32)
            an = jnp.zeros((L,), jnp.float32)
            for k in range(EMBED // L):
                sl = pl.ds(k * L, L)
                pv = path_v[i, sl]
                ap = ap + pv * pos_v[i, sl]
                an = an + pv * neg_v[i, sl]
            lane = iota == r
            posvec = jnp.where(lane, fold(ap), posvec)
            negvec = jnp.where(lane, fold(an), negvec)
        out_sl = pl.ds(g * L, L)
        pos_s[out_sl] = posvec
        neg_s[out_sl] = negvec
        return carry

    lax.fori_loop(0, ROWS // L, group, 0)

    pltpu.sync_copy(pos_s, pos_out_hbm.at[pl.ds(base, ROWS)])
    pltpu.sync_copy(neg_s, neg_out_hbm.at[pl.ds(base, ROWS)])


def kernel(ents_path_idxs, embeddings_entities, embeddings_paths):
    idx = ents_path_idxs.astype(jnp.int32)
    pidx = idx[:, 0].reshape(NW, NCHUNK, CHUNK)
    aidx = idx[:, 1].reshape(NW, NCHUNK, CHUNK)
    bidx = idx[:, 2].reshape(NW, NCHUNK, CHUNK)

    mesh = plsc.VectorSubcoreMesh(core_axis_name="c", subcore_axis_name="s",
                                  num_cores=NC, num_subcores=NS)
    run = pl.kernel(
        _body,
        out_type=[jax.ShapeDtypeStruct((B,), jnp.float32),
                  jax.ShapeDtypeStruct((B,), jnp.float32)],
        mesh=mesh,
        compiler_params=pltpu.CompilerParams(use_tc_tiling_on_sc=False),
        scratch_types=[
            pltpu.VMEM((NCHUNK, CHUNK), jnp.int32),
            pltpu.VMEM((NCHUNK, CHUNK), jnp.int32),
            pltpu.VMEM((NCHUNK, CHUNK), jnp.int32),
            pltpu.VMEM((ROWS, EMBED), jnp.float32),
            pltpu.VMEM((ROWS, EMBED), jnp.float32),
            pltpu.VMEM((ROWS, EMBED), jnp.float32),
            pltpu.VMEM((ROWS,), jnp.float32),
            pltpu.VMEM((ROWS,), jnp.float32),
            pltpu.SemaphoreType.DMA,
        ],
    )
    pos, neg = run(pidx, aidx, bidx, embeddings_paths, embeddings_entities)
    return pos.reshape(B, 1), neg.reshape(B, 1)
